# baseline (speedup 1.0000x reference)
import jax
import jax.numpy as jnp
from jax.experimental import pallas as pl
from jax.experimental.pallas import tpu as pltpu
from jax.experimental.pallas import tpu_sc as plsc

BATCH = 16384
FIELDS = 26
DIM = 16
D_IN = FIELDS * DIM
D_PAD = 512
NUM_SLOTS = BATCH * D_PAD // DIM
T_ROWS = BATCH * D_PAD // 128

GATHER_WINDOW = 128

BB = 2048


def _sc_gather_tiled(table, perm_idx):
    mesh = plsc.VectorSubcoreMesh(core_axis_name="core", subcore_axis_name="subcore")

    @pl.kernel(
        out_type=jax.ShapeDtypeStruct((T_ROWS, 128), table.dtype),
        mesh=mesh,
        scratch_types=[pltpu.VMEM((GATHER_WINDOW, DIM), table.dtype)],
        compiler_params=pltpu.CompilerParams(use_tc_tiling_on_sc=False),
    )
    def gather_kernel(tab_hbm, idx_hbm, out_hbm, rows_v):
        def body(idx_vmem, out_vmem):
            pltpu.sync_copy(tab_hbm.at[idx_vmem.at[0]], rows_v)

            @pl.loop(0, GATHER_WINDOW // 8)
            def _(r):
                for k in range(8):
                    out_vmem[r, pl.ds(16 * k, 16)] = rows_v[8 * r + k, :]

        pltpu.emit_pipeline(
            body,
            grid=(NUM_SLOTS // GATHER_WINDOW,),
            in_specs=[pl.BlockSpec((1, GATHER_WINDOW), index_map=lambda i: (0, i))],
            out_specs=[
                pl.BlockSpec((GATHER_WINDOW * DIM // 128, 128), index_map=lambda i: (i, 0))
            ],
            core_axis_name=("core", "subcore"),
            dimension_semantics=(pltpu.PARALLEL,),
        )(idx_hbm, out_hbm)

    return gather_kernel(table, perm_idx)


def _mlp_block(t_ref, w1, b1, w2, b2, w3, b3, w4, b4, w5, b5, out_ref):
    t = t_ref[...].reshape(BB // 8, 4, 8, 128)
    h = jnp.concatenate(
        [t[:, j].reshape(BB, 128) for j in range(4)], axis=1
    )
    h = jnp.maximum(jnp.dot(h, w1[...], preferred_element_type=jnp.float32) + b1[...], 0.0)
    h = jnp.maximum(jnp.dot(h, w2[...], preferred_element_type=jnp.float32) + b2[...], 0.0)
    h = jnp.maximum(jnp.dot(h, w3[...], preferred_element_type=jnp.float32) + b3[...], 0.0)
    h = jnp.maximum(jnp.dot(h, w4[...], preferred_element_type=jnp.float32) + b4[...], 0.0)
    o = jnp.dot(h, w5[...], preferred_element_type=jnp.float32) + b5[...]
    out_ref[...] = jax.nn.sigmoid(o)


def _mlp(tarr, W1p, b1, W2, b2, W3, b3, W4, b4, W5, b5):
    full = lambda a: pl.BlockSpec(a.shape, lambda i: (0,) * a.ndim)
    return pl.pallas_call(
        _mlp_block,
        grid=(BATCH // BB,),
        in_specs=[
            pl.BlockSpec((BB * 4, 128), lambda i: (i, 0)),
            full(W1p), full(b1), full(W2), full(b2), full(W3), full(b3),
            full(W4), full(b4), full(W5), full(b5),
        ],
        out_specs=pl.BlockSpec((BB, 1), lambda i: (i, 0)),
        out_shape=jax.ShapeDtypeStruct((BATCH, 1), jnp.float32),
    )(tarr, W1p, b1, W2, b2, W3, b3, W4, b4, W5, b5)


def kernel(x, table, W1, b1, W2, b2, W3, b3, W4, b4, W5, b5):
    xpad = jnp.pad(x, ((0, 0), (0, 32 - FIELDS)))
    perm_idx = (
        xpad.reshape(BATCH // 8, 8, 4, 8)
        .transpose(0, 2, 1, 3)
        .reshape(1, NUM_SLOTS)
    )
    tarr = _sc_gather_tiled(table, perm_idx)
    W1p = jnp.zeros((D_PAD, 512), jnp.float32).at[:D_IN].set(W1)
    return _mlp(
        tarr,
        W1p, b1.reshape(1, -1),
        W2, b2.reshape(1, -1),
        W3, b3.reshape(1, -1),
        W4, b4.reshape(1, -1),
        W5, b5.reshape(1, -1),
    )


# speedup vs baseline: 1.6367x; 1.6367x over previous
import jax
import jax.numpy as jnp
from jax.experimental import pallas as pl
from jax.experimental.pallas import tpu as pltpu
from jax.experimental.pallas import tpu_sc as plsc

BATCH = 16384
FIELDS = 26
DIM = 16
NUM_IDX = BATCH * FIELDS

GATHER_WINDOW = 128
IDX_ROWS = NUM_IDX // GATHER_WINDOW

BB = 2048


def _sc_gather(table, idx2d):
    mesh = plsc.VectorSubcoreMesh(core_axis_name="core", subcore_axis_name="subcore")

    @pl.kernel(
        out_type=jax.ShapeDtypeStruct((NUM_IDX, DIM), table.dtype),
        mesh=mesh,
        compiler_params=pltpu.CompilerParams(use_tc_tiling_on_sc=False),
    )
    def gather_kernel(tab_hbm, idx_hbm, out_hbm):
        def body(idx_vmem, out_vmem):
            pltpu.sync_copy(tab_hbm.at[idx_vmem.at[0]], out_vmem)

        pltpu.emit_pipeline(
            body,
            grid=(IDX_ROWS,),
            in_specs=[pl.BlockSpec((1, GATHER_WINDOW), index_map=lambda i: (i, 0))],
            out_specs=[pl.BlockSpec((GATHER_WINDOW, DIM), index_map=lambda i: (i, 0))],
            core_axis_name=("core", "subcore"),
            dimension_semantics=(pltpu.PARALLEL,),
        )(idx_hbm, out_hbm)

    return gather_kernel(table, idx2d)


def _mlp_block(emb_ref, w1, b1, w2, b2, w3, b3, w4, b4, w5, b5, out_ref):
    h = emb_ref[...]
    h = jnp.maximum(jnp.dot(h, w1[...], preferred_element_type=jnp.float32) + b1[...], 0.0)
    h = jnp.maximum(jnp.dot(h, w2[...], preferred_element_type=jnp.float32) + b2[...], 0.0)
    h = jnp.maximum(jnp.dot(h, w3[...], preferred_element_type=jnp.float32) + b3[...], 0.0)
    h = jnp.maximum(jnp.dot(h, w4[...], preferred_element_type=jnp.float32) + b4[...], 0.0)
    o = jnp.dot(h, w5[...], preferred_element_type=jnp.float32) + b5[...]
    out_ref[...] = jax.nn.sigmoid(o)


def _mlp(emb, W1, b1, W2, b2, W3, b3, W4, b4, W5, b5):
    full = lambda a: pl.BlockSpec(a.shape, lambda i: (0,) * a.ndim)
    return pl.pallas_call(
        _mlp_block,
        grid=(BATCH // BB,),
        in_specs=[
            pl.BlockSpec((BB, FIELDS * DIM), lambda i: (i, 0)),
            full(W1), full(b1), full(W2), full(b2), full(W3), full(b3),
            full(W4), full(b4), full(W5), full(b5),
        ],
        out_specs=pl.BlockSpec((BB, 1), lambda i: (i, 0)),
        out_shape=jax.ShapeDtypeStruct((BATCH, 1), jnp.float32),
    )(emb, W1, b1, W2, b2, W3, b3, W4, b4, W5, b5)


def kernel(x, table, W1, b1, W2, b2, W3, b3, W4, b4, W5, b5):
    idx2d = x.reshape(IDX_ROWS, GATHER_WINDOW)
    rows = _sc_gather(table, idx2d)
    emb = rows.reshape(BATCH, FIELDS * DIM)
    return _mlp(
        emb,
        W1, b1.reshape(1, -1),
        W2, b2.reshape(1, -1),
        W3, b3.reshape(1, -1),
        W4, b4.reshape(1, -1),
        W5, b5.reshape(1, -1),
    )
